# baseline (speedup 1.0000x reference)
import functools

import jax
import jax.numpy as jnp
from jax import lax
from jax.experimental import pallas as pl
from jax.experimental.pallas import tpu as pltpu
from jax.experimental.pallas import tpu_sc as plsc

L = 16
CHUNK = 128
GROUPS = CHUNK // L


def _sc_body(n_chunks, ew, src_hbm, dst_hbm, typ_hbm, ent_hbm, rel_hbm,
             out_hbm, idx_s, idx_d, idx_r, rows_s, rows_d, rows_r, outb,
             sem_s, sem_d, sem_r):
    nc = 2
    wid = lax.axis_index("s") * nc + lax.axis_index("c")
    base0 = wid * ew
    iota = lax.iota(jnp.int32, L)

    def chunk_body(c, carry):
        base = base0 + c * CHUNK
        pltpu.sync_copy(src_hbm.at[pl.ds(base, CHUNK)], idx_s)
        pltpu.sync_copy(dst_hbm.at[pl.ds(base, CHUNK)], idx_d)
        pltpu.sync_copy(typ_hbm.at[pl.ds(base, CHUNK)], idx_r)
        cp_s = pltpu.async_copy(ent_hbm.at[idx_s], rows_s, sem_s)
        cp_d = pltpu.async_copy(ent_hbm.at[idx_d], rows_d, sem_d)
        cp_r = pltpu.async_copy(rel_hbm.at[idx_r], rows_r, sem_r)
        cp_s.wait()
        cp_d.wait()
        cp_r.wait()

        for g in range(GROUPS):
            row = iota + (g * L)

            def dim_body(j, acc):
                for k in range(8):
                    col = jnp.full((L,), 0, jnp.int32) + (j * 8 + k)
                    s = plsc.load_gather(rows_s, [row, col])
                    d = plsc.load_gather(rows_d, [row, col])
                    r = plsc.load_gather(rows_r, [row, col])
                    acc = acc + (s * r) * d
                return acc

            acc = lax.fori_loop(0, 32, dim_body, jnp.zeros((L,), jnp.float32))
            outb[pl.ds(g * L, L)] = 1.0 / (1.0 + jnp.exp(-acc))

        pltpu.sync_copy(outb, out_hbm.at[pl.ds(base, CHUNK)])
        return carry

    lax.fori_loop(0, n_chunks, chunk_body, 0)


@functools.partial(jax.jit, static_argnames=("e_pad",))
def _dist_mult_sc(src, dst, typ, ent, rel, e_pad):
    info = plsc.get_sparse_core_info()
    nw = info.num_cores * info.num_subcores
    ew = e_pad // nw
    n_chunks = ew // CHUNK
    mesh = plsc.VectorSubcoreMesh(core_axis_name="c", subcore_axis_name="s")
    kfn = pl.kernel(
        functools.partial(_sc_body, n_chunks, ew),
        out_type=jax.ShapeDtypeStruct((e_pad,), jnp.float32),
        mesh=mesh,
        scratch_types=[
            pltpu.VMEM((CHUNK,), jnp.int32),
            pltpu.VMEM((CHUNK,), jnp.int32),
            pltpu.VMEM((CHUNK,), jnp.int32),
            pltpu.VMEM((CHUNK, 256), jnp.float32),
            pltpu.VMEM((CHUNK, 256), jnp.float32),
            pltpu.VMEM((CHUNK, 256), jnp.float32),
            pltpu.VMEM((CHUNK,), jnp.float32),
            pltpu.SemaphoreType.DMA,
            pltpu.SemaphoreType.DMA,
            pltpu.SemaphoreType.DMA,
        ],
        compiler_params=pltpu.CompilerParams(use_tc_tiling_on_sc=False,
                                             needs_layout_passes=False),
    )
    return kfn(src, dst, typ, ent, rel)


def kernel(edge_index, edge_type, entity_embedding, relation_embedding):
    e = edge_type.shape[0]
    quantum = 32 * CHUNK
    e_pad = ((e + quantum - 1) // quantum) * quantum
    pad = e_pad - e
    src = jnp.concatenate([edge_index[0], jnp.zeros((pad,), jnp.int32)])
    dst = jnp.concatenate([edge_index[1], jnp.zeros((pad,), jnp.int32)])
    typ = jnp.concatenate([edge_type, jnp.zeros((pad,), jnp.int32)])
    out = _dist_mult_sc(src, dst, typ, entity_embedding, relation_embedding,
                        e_pad)
    return out[:e]


# speedup vs baseline: 4.8477x; 4.8477x over previous
import functools

import jax
import jax.numpy as jnp
from jax import lax
from jax.experimental import pallas as pl
from jax.experimental.pallas import tpu as pltpu
from jax.experimental.pallas import tpu_sc as plsc

L = 16
CHUNK = 64
GROUPS = CHUNK // L


def _sc_body(n_chunks, ew, src_hbm, dst_hbm, typ_hbm, ent_hbm, rel_hbm,
             out_hbm, idx_s, idx_d, idx_r, rows_s, rows_d, rows_r, outb,
             sem_s, sem_d, sem_r, sem_o, sem_i):
    nc = 2
    wid = lax.axis_index("s") * nc + lax.axis_index("c")
    base0 = wid * ew
    iota = lax.iota(jnp.int32, L)

    ci = pltpu.async_copy(src_hbm.at[pl.ds(base0, ew)], idx_s, sem_i)
    cd = pltpu.async_copy(dst_hbm.at[pl.ds(base0, ew)], idx_d, sem_i)
    cr = pltpu.async_copy(typ_hbm.at[pl.ds(base0, ew)], idx_r, sem_i)
    ci.wait()
    cd.wait()
    cr.wait()

    def issue(c, b):
        off = c * CHUNK
        pltpu.async_copy(ent_hbm.at[idx_s.at[pl.ds(off, CHUNK)]],
                         rows_s.at[b], sem_s.at[b])
        pltpu.async_copy(ent_hbm.at[idx_d.at[pl.ds(off, CHUNK)]],
                         rows_d.at[b], sem_d.at[b])
        pltpu.async_copy(rel_hbm.at[idx_r.at[pl.ds(off, CHUNK)]],
                         rows_r.at[b], sem_r.at[b])

    def drain(b):
        pltpu.make_async_copy(ent_hbm.at[idx_s.at[pl.ds(0, CHUNK)]],
                              rows_s.at[b], sem_s.at[b]).wait()
        pltpu.make_async_copy(ent_hbm.at[idx_d.at[pl.ds(0, CHUNK)]],
                              rows_d.at[b], sem_d.at[b]).wait()
        pltpu.make_async_copy(rel_hbm.at[idx_r.at[pl.ds(0, CHUNK)]],
                              rows_r.at[b], sem_r.at[b]).wait()

    def drain_out(c, b):
        pltpu.make_async_copy(
            outb.at[b], out_hbm.at[pl.ds(base0 + c * CHUNK, CHUNK)],
            sem_o.at[b]).wait()

    issue(0, 0)

    def chunk_pair(i2, carry):
        for b in range(2):
            c = i2 * 2 + b
            drain(b)

            @pl.when(c + 1 < n_chunks)
            def _():
                issue(c + 1, 1 - b)

            @pl.when(c >= 2)
            def _():
                drain_out(c - 2, b)

            rs, rd, rr = rows_s.at[b], rows_d.at[b], rows_r.at[b]
            for g in range(GROUPS):
                row = iota + (g * L)

                def dim_body(j, acc, rs=rs, rd=rd, rr=rr, row=row):
                    for k in range(8):
                        col = (iota + (j * 8 + k)) & 255
                        s = plsc.load_gather(rs, [row, col])
                        d = plsc.load_gather(rd, [row, col])
                        r = plsc.load_gather(rr, [row, col])
                        acc = acc + (s * r) * d
                    return acc

                acc = lax.fori_loop(0, 32, dim_body,
                                    jnp.zeros((L,), jnp.float32))
                outb[b, pl.ds(g * L, L)] = 1.0 / (1.0 + jnp.exp(-acc))
            pltpu.async_copy(outb.at[b],
                             out_hbm.at[pl.ds(base0 + c * CHUNK, CHUNK)],
                             sem_o.at[b])
        return carry

    lax.fori_loop(0, n_chunks // 2, chunk_pair, 0)
    drain_out(n_chunks - 2, 0)
    drain_out(n_chunks - 1, 1)


@functools.partial(jax.jit, static_argnames=("e_pad",))
def _dist_mult_sc(src, dst, typ, ent, rel, e_pad):
    info = plsc.get_sparse_core_info()
    nw = info.num_cores * info.num_subcores
    ew = e_pad // nw
    n_chunks = ew // CHUNK
    mesh = plsc.VectorSubcoreMesh(core_axis_name="c", subcore_axis_name="s")
    kfn = pl.kernel(
        functools.partial(_sc_body, n_chunks, ew),
        out_type=jax.ShapeDtypeStruct((e_pad,), jnp.float32),
        mesh=mesh,
        scratch_types=[
            pltpu.VMEM((ew,), jnp.int32),
            pltpu.VMEM((ew,), jnp.int32),
            pltpu.VMEM((ew,), jnp.int32),
            pltpu.VMEM((2, CHUNK, 256), jnp.float32),
            pltpu.VMEM((2, CHUNK, 256), jnp.float32),
            pltpu.VMEM((2, CHUNK, 256), jnp.float32),
            pltpu.VMEM((2, CHUNK), jnp.float32),
            pltpu.SemaphoreType.DMA((2,)),
            pltpu.SemaphoreType.DMA((2,)),
            pltpu.SemaphoreType.DMA((2,)),
            pltpu.SemaphoreType.DMA((2,)),
            pltpu.SemaphoreType.DMA,
        ],
        compiler_params=pltpu.CompilerParams(use_tc_tiling_on_sc=False,
                                             needs_layout_passes=False),
    )
    return kfn(src, dst, typ, ent, rel)


def kernel(edge_index, edge_type, entity_embedding, relation_embedding):
    e = edge_type.shape[0]
    quantum = 32 * CHUNK * 2
    e_pad = ((e + quantum - 1) // quantum) * quantum
    pad = e_pad - e
    src = jnp.concatenate([edge_index[0], jnp.zeros((pad,), jnp.int32)])
    dst = jnp.concatenate([edge_index[1], jnp.zeros((pad,), jnp.int32)])
    typ = jnp.concatenate([edge_type, jnp.zeros((pad,), jnp.int32)])
    out = _dist_mult_sc(src, dst, typ, entity_embedding, relation_embedding,
                        e_pad)
    return out[:e]


# speedup vs baseline: 7.7378x; 1.5962x over previous
import functools

import jax
import jax.numpy as jnp
from jax import lax
from jax.experimental import pallas as pl
from jax.experimental.pallas import tpu as pltpu
from jax.experimental.pallas import tpu_sc as plsc

L = 16
CHUNK = 48
GROUPS = CHUNK // L
EDGES = 300000
NW = 32
EW = 9504
EW_LAST = EDGES - (NW - 1) * EW


def _sc_body(src_hbm, dst_hbm, typ_hbm, ent_hbm, rel_hbm, out_hbm,
             idx_s, idx_d, idx_r, rows_s, rows_d, rows_r, outb,
             sem_s, sem_d, sem_r, sem_i):
    nc = 2
    wid = lax.axis_index("s") * nc + lax.axis_index("c")
    base0 = wid * EW
    last = wid == NW - 1
    n_chunks = jnp.where(last, EW_LAST // CHUNK, EW // CHUNK)
    iota = lax.iota(jnp.int32, L)

    @pl.when(jnp.logical_not(last))
    def _():
        ci = pltpu.async_copy(src_hbm.at[pl.ds(base0, EW)],
                              idx_s.at[pl.ds(0, EW)], sem_i)
        cd = pltpu.async_copy(dst_hbm.at[pl.ds(base0, EW)],
                              idx_d.at[pl.ds(0, EW)], sem_i)
        cr = pltpu.async_copy(typ_hbm.at[pl.ds(base0, EW)],
                              idx_r.at[pl.ds(0, EW)], sem_i)
        ci.wait()
        cd.wait()
        cr.wait()

    @pl.when(last)
    def _():
        ci = pltpu.async_copy(src_hbm.at[pl.ds(base0, EW_LAST)],
                              idx_s.at[pl.ds(0, EW_LAST)], sem_i)
        cd = pltpu.async_copy(dst_hbm.at[pl.ds(base0, EW_LAST)],
                              idx_d.at[pl.ds(0, EW_LAST)], sem_i)
        cr = pltpu.async_copy(typ_hbm.at[pl.ds(base0, EW_LAST)],
                              idx_r.at[pl.ds(0, EW_LAST)], sem_i)
        ci.wait()
        cd.wait()
        cr.wait()

    def issue(c, b):
        off = c * CHUNK
        pltpu.async_copy(ent_hbm.at[idx_s.at[pl.ds(off, CHUNK)]],
                         rows_s.at[b], sem_s.at[b])
        pltpu.async_copy(ent_hbm.at[idx_d.at[pl.ds(off, CHUNK)]],
                         rows_d.at[b], sem_d.at[b])
        pltpu.async_copy(rel_hbm.at[idx_r.at[pl.ds(off, CHUNK)]],
                         rows_r.at[b], sem_r.at[b])

    def drain(b):
        pltpu.make_async_copy(ent_hbm.at[idx_s.at[pl.ds(0, CHUNK)]],
                              rows_s.at[b], sem_s.at[b]).wait()
        pltpu.make_async_copy(ent_hbm.at[idx_d.at[pl.ds(0, CHUNK)]],
                              rows_d.at[b], sem_d.at[b]).wait()
        pltpu.make_async_copy(rel_hbm.at[idx_r.at[pl.ds(0, CHUNK)]],
                              rows_r.at[b], sem_r.at[b]).wait()

    issue(0, 0)

    def chunk_pair(i2, carry):
        for b in range(2):
            c = i2 * 2 + b

            @pl.when(c < n_chunks)
            def _(c=c, b=b):
                drain(b)

                @pl.when(c + 1 < n_chunks)
                def _():
                    issue(c + 1, 1 - b)

                rs, rd, rr = rows_s.at[b], rows_d.at[b], rows_r.at[b]
                for g in range(GROUPS):
                    row = iota + (g * L)

                    def dim_body(j, acc, rs=rs, rd=rd, rr=rr, row=row):
                        for k in range(8):
                            col = (iota + (j * 8 + k)) & 255
                            s = plsc.load_gather(rs, [row, col])
                            d = plsc.load_gather(rd, [row, col])
                            r = plsc.load_gather(rr, [row, col])
                            acc = acc + (s * r) * d
                        return acc

                    acc = lax.fori_loop(0, 32, dim_body,
                                        jnp.zeros((L,), jnp.float32))
                    outb[pl.ds(c * CHUNK + g * L, L)] = (
                        1.0 / (1.0 + jnp.exp(-acc)))
        return carry

    lax.fori_loop(0, EW // CHUNK // 2, chunk_pair, 0)

    @pl.when(jnp.logical_not(last))
    def _():
        pltpu.sync_copy(outb.at[pl.ds(0, EW)], out_hbm.at[pl.ds(base0, EW)])

    @pl.when(last)
    def _():
        pltpu.sync_copy(outb.at[pl.ds(0, EW_LAST)],
                        out_hbm.at[pl.ds(base0, EW_LAST)])


@jax.jit
def _dist_mult_sc(src, dst, typ, ent, rel):
    mesh = plsc.VectorSubcoreMesh(core_axis_name="c", subcore_axis_name="s")
    kfn = pl.kernel(
        _sc_body,
        out_type=jax.ShapeDtypeStruct((EDGES,), jnp.float32),
        mesh=mesh,
        scratch_types=[
            pltpu.VMEM((EW,), jnp.int32),
            pltpu.VMEM((EW,), jnp.int32),
            pltpu.VMEM((EW,), jnp.int32),
            pltpu.VMEM((2, CHUNK, 256), jnp.float32),
            pltpu.VMEM((2, CHUNK, 256), jnp.float32),
            pltpu.VMEM((2, CHUNK, 256), jnp.float32),
            pltpu.VMEM((EW,), jnp.float32),
            pltpu.SemaphoreType.DMA((2,)),
            pltpu.SemaphoreType.DMA((2,)),
            pltpu.SemaphoreType.DMA((2,)),
            pltpu.SemaphoreType.DMA,
        ],
        compiler_params=pltpu.CompilerParams(use_tc_tiling_on_sc=False,
                                             needs_layout_passes=False),
    )
    return kfn(src, dst, typ, ent, rel)


def kernel(edge_index, edge_type, entity_embedding, relation_embedding):
    return _dist_mult_sc(edge_index[0], edge_index[1], edge_type,
                         entity_embedding, relation_embedding)
